# Initial kernel scaffold; baseline (speedup 1.0000x reference)
import jax
import jax.numpy as jnp
from jax.experimental import pallas as pl


def kernel(samples):
    raise NotImplementedError("write your pallas kernel here")


# speedup vs baseline: 3.1912x; 3.1912x over previous
import functools

import jax
import jax.numpy as jnp
from jax import lax
from jax.experimental import pallas as pl
from jax.experimental.pallas import tpu as pltpu
from jax.experimental.pallas import tpu_sc as plsc

N = 16384
D = 256
L = 16
NV = N // L
NC, NS = 2, 16
NW = NC * NS
CPW = D // NW
RBITS = (11, 11, 10)
RSHIFT = (0, 11, 22)
NBINS_MAX = 1 << 11
INV_N1 = 1.0 / (N + 1)


def _digits(k_i32, shift, mask):
  ku = plsc.bitcast(k_i32, jnp.uint32)
  d = (ku >> shift) & mask
  return plsc.bitcast(d, jnp.int32)


def _emit_pass(src_k, src_v, dst_k, dst_v, hist, fout, pass_idx):
  shift = RSHIFT[pass_idx]
  mask = jnp.uint32((1 << RBITS[pass_idx]) - 1)
  nbins = 1 << RBITS[pass_idx]
  first = pass_idx == 0
  last = pass_idx == len(RBITS) - 1

  def zero_body(b, carry):
    hist[pl.ds(b * L, L)] = jnp.zeros((L,), jnp.int32)
    return carry

  lax.fori_loop(0, nbins // L, zero_body, 0, unroll=4)

  def hist_body(i, carry):
    k = src_k[pl.ds(i * L, L)]
    if first:
      m = jnp.right_shift(k, 31)
      k = k ^ (m | jnp.int32(-(2 ** 31)))
      src_k[pl.ds(i * L, L)] = k
    d = _digits(k, shift, mask)
    cnt, last_m = plsc.scan_count(d)
    base = plsc.load_gather(hist, [d])
    plsc.store_scatter(hist, [d], base + cnt, mask=last_m)
    return carry

  lax.fori_loop(0, NV, hist_body, 0, unroll=2)

  def scan_body(b, carry):
    v = hist[pl.ds(b * L, L)]
    s = plsc.cumsum(v)
    hist[pl.ds(b * L, L)] = s - v + carry
    return carry + jnp.sum(v)

  lax.fori_loop(0, nbins // L, scan_body, jnp.int32(0), unroll=2)

  def perm_body(i, carry):
    k = src_k[pl.ds(i * L, L)]
    d = _digits(k, shift, mask)
    cnt, last_m = plsc.scan_count(d)
    base = plsc.load_gather(hist, [d])
    pos = base + cnt - 1
    if first:
      v = lax.iota(jnp.int32, L) + i * L
    else:
      v = src_v[pl.ds(i * L, L)]
    if last:
      plsc.store_scatter(
          fout, [v], (pos + 1).astype(jnp.float32) * INV_N1)
    else:
      plsc.store_scatter(dst_k, [pos], k)
      plsc.store_scatter(dst_v, [pos], v)
    plsc.store_scatter(hist, [d], base + cnt, mask=last_m)
    return carry

  lax.fori_loop(0, NV, perm_body, 0, unroll=2)


def _rank_body(st_hbm, out_hbm, k0, k1, v0, v1, hist, fout):
  wid = lax.axis_index("s") * NC + lax.axis_index("c")

  def col_body(ci, carry):
    col = wid * CPW + ci
    pltpu.sync_copy(st_hbm.at[col], k0)
    _emit_pass(k0, None, k1, v1, hist, fout, 0)
    _emit_pass(k1, v1, k0, v0, hist, fout, 1)
    _emit_pass(k0, v0, None, None, hist, fout, 2)
    pltpu.sync_copy(fout, out_hbm.at[col])
    return carry

  lax.fori_loop(0, CPW, col_body, 0)


@jax.jit
def _rank_columns(st_keys):
  mesh = plsc.VectorSubcoreMesh(
      core_axis_name="c", subcore_axis_name="s",
      num_cores=NC, num_subcores=NS)
  f = pl.kernel(
      _rank_body,
      out_type=jax.ShapeDtypeStruct((D, N), jnp.float32),
      mesh=mesh,
      compiler_params=pltpu.CompilerParams(needs_layout_passes=False),
      scratch_types=[
          pltpu.VMEM((N,), jnp.int32),
          pltpu.VMEM((N,), jnp.int32),
          pltpu.VMEM((N,), jnp.int32),
          pltpu.VMEM((N,), jnp.int32),
          pltpu.VMEM((NBINS_MAX,), jnp.int32),
          pltpu.VMEM((N,), jnp.float32),
      ],
  )
  return f(st_keys)


def kernel(samples):
  st = jnp.transpose(samples).view(jnp.int32)
  ranks = _rank_columns(st)
  return ranks[None, :, :]
